# baseline (speedup 1.0000x reference)
import functools

import jax
import jax.numpy as jnp
from jax import lax
from jax.experimental import pallas as pl
from jax.experimental.pallas import tpu as pltpu
from jax.experimental.pallas import tpu_sc as plsc

_NC = 2
_NS = 16
_NW = _NC * _NS
_CH = 128
_L = 16

_TBLK = 8192


def _transpose_tc(tbl_t, v, d):
    grid = (v + _TBLK - 1) // _TBLK
    half = _TBLK // 2

    def body(in_ref, out_ref):
        out_ref[:, :d] = in_ref[:, :half].T
        out_ref[:, d:] = in_ref[:, half:].T

    return pl.pallas_call(
        body,
        grid=(grid,),
        in_specs=[pl.BlockSpec((d, _TBLK), lambda i: (0, i))],
        out_specs=pl.BlockSpec((half, 2 * d), lambda i: (i, 0)),
        out_shape=jax.ShapeDtypeStruct((grid * half, 2 * d), jnp.float32),
    )(tbl_t)


def _emb_lookup(idx, par, table, seq, d):
    mesh = plsc.VectorSubcoreMesh(core_axis_name="c", subcore_axis_name="s")

    @functools.partial(
        pl.kernel,
        mesh=mesh,
        out_type=jax.ShapeDtypeStruct((seq, d, _NW * _CH), jnp.float32),
        scratch_types=[
            pltpu.VMEM((seq, _CH), jnp.int32),
            pltpu.VMEM((seq, _CH), jnp.int32),
            pltpu.VMEM((2, _CH, 128), jnp.float32),
            pltpu.VMEM((2, d, _CH), jnp.float32),
            pltpu.SemaphoreType.DMA,
            pltpu.SemaphoreType.DMA,
            pltpu.SemaphoreType.DMA,
            pltpu.SemaphoreType.DMA,
        ],
        compiler_params=pltpu.CompilerParams(
            use_tc_tiling_on_sc=True, needs_layout_passes=False),
    )
    def body(idx_hbm, par_hbm, tbl_hbm, out_hbm, idx_v, par_v, rows_v, blk_v,
             g0, g1, o0, o1):
        wid = lax.axis_index("s") * _NC + lax.axis_index("c")
        lane0 = wid * _CH
        pltpu.sync_copy(idx_hbm.at[wid], idx_v)
        pltpu.sync_copy(par_hbm.at[wid], par_v)
        gsems = (g0, g1)
        osems = (o0, o1)

        def start_gather(k, b, sem):
            pltpu.async_copy(tbl_hbm.at[idx_v.at[k]], rows_v.at[b], sem)

        def wait_gather(b, sem):
            pltpu.make_async_copy(
                tbl_hbm.at[pl.ds(0, _CH)], rows_v.at[b], sem).wait()

        def start_out(k, b, sem):
            pltpu.async_copy(
                blk_v.at[b], out_hbm.at[k, :, pl.ds(lane0, _CH)], sem)

        def wait_out(b, sem):
            pltpu.make_async_copy(
                blk_v.at[b], out_hbm.at[0, :, pl.ds(lane0, _CH)], sem).wait()

        def assemble(b, k):
            rows = rows_v.at[b]
            blk = blk_v.at[b]
            kfull = jnp.full((_L,), k, jnp.int32)

            @plsc.parallel_loop(0, _CH, 1, unroll=8)
            def _(j):
                jfull = jnp.full((_L,), j, jnp.int32)
                poff = plsc.load_gather(par_v, [kfull, jfull]) * d
                for u in range(d // _L):
                    cidx = lax.iota(jnp.int32, _L) + _L * u
                    vals = plsc.load_gather(rows, [jfull, cidx + poff])
                    plsc.store_scatter(blk, [cidx, jfull], vals)

        start_gather(0, 0, gsems[0])

        def step(j, carry):
            for b in range(2):
                k = 2 * j + b
                o = 1 - b

                @pl.when(k + 1 < seq)
                def _():
                    start_gather(k + 1, o, gsems[o])

                wait_gather(b, gsems[b])

                @pl.when(k >= 2)
                def _():
                    wait_out(b, osems[b])

                assemble(b, k)
                start_out(k, b, osems[b])
            return carry

        lax.fori_loop(0, seq // 2, step, 0)
        wait_out(0, osems[0])
        wait_out(1, osems[1])

    return body(idx, par, table)


def kernel(inputs, embeddings):
    b, s = inputs.shape
    v, d = embeddings.shape
    blk_sh = _TBLK.bit_length() - 1
    p = inputs & (_TBLK - 1)
    staged = ((inputs >> blk_sh) << (blk_sh - 1)) | (p & (_TBLK // 2 - 1))
    idx = staged.reshape(_NW, _CH, s).transpose(0, 2, 1)
    par = (p >> (blk_sh - 1)).reshape(_NW, _CH, s).transpose(0, 2, 1)
    table = _transpose_tc(embeddings.T, v, d)
    out = _emb_lookup(idx, par, table, s, d)
    return jnp.transpose(out, (2, 0, 1))
